# baseline (speedup 1.0000x reference)
import functools

import numpy as np
import jax
import jax.numpy as jnp
from jax import lax
from jax.experimental import pallas as pl
from jax.experimental.pallas import tpu as pltpu
from jax.experimental.pallas import tpu_sc as plsc

EMBED = 64
PADE = 128
SEQ = 200
BATCH = 1024
MAX_LEN = 512

NC, NS = 2, 16
NW = NC * NS
N = BATCH * SEQ
NPW = N // NW
SUB = 128
CH = 256
NCHUNK = NPW // CH
KSUB = CH // SUB


def _make_pe_np(max_len, d_model):
    position = np.arange(max_len, dtype=np.float32)[:, None]
    div_term = np.exp(
        np.arange(0, d_model, 2, dtype=np.float32) * -(np.log(10000.0) / d_model)
    )
    pe = np.zeros((max_len, d_model), dtype=np.float32)
    pe[:, 0::2] = np.sin(position * div_term)
    pe[:, 1::2] = np.cos(position * div_term)
    return pe


_PE = _make_pe_np(MAX_LEN, EMBED)[:SEQ]


TB = 512
NRB = TB // 2
NBF = 1000000 // TB
TAIL_T0 = NBF * TB
TAIL_N = 1000000 - TAIL_T0


def _tr_body(tokt, tailp, tokc, blk_v, rowb_v):
    c = lax.axis_index("c")
    s = lax.axis_index("s")
    wid = s * NC + c
    lane = lax.iota(jnp.int32, 16)

    def _transpose_block(nrows):
        @pl.loop(0, nrows)
        def _t(rp):
            for u in range(PADE // 16):
                e0 = (u % 4) * 16
                tloc = 2 * rp + (u // 4)
                src = plsc.load_gather(blk_v, [e0 + lane, lane * 0 + tloc])
                rowb_v[rp, pl.ds(u * 16, 16)] = src

    @pl.loop(0, (NBF + NW - 1) // NW)
    def _blk(bi):
        b = wid + bi * NW

        @pl.when(b < NBF)
        def _():
            t0 = pl.multiple_of(b * TB, TB)
            pltpu.sync_copy(tokt.at[:, pl.ds(t0, TB)], blk_v)
            _transpose_block(NRB)
            orow = pl.multiple_of(b * NRB, NRB)
            pltpu.sync_copy(rowb_v, tokc.at[pl.ds(orow, NRB)])

    @pl.when(wid == 0)
    def _tail():
        pltpu.sync_copy(tailp, blk_v.at[:, pl.ds(0, 128)])
        _transpose_block(TAIL_N // 2)
        pltpu.sync_copy(rowb_v.at[pl.ds(0, TAIL_N // 2)],
                        tokc.at[pl.ds(TAIL_T0 // 2, TAIL_N // 2)])


@jax.jit
def _tr_call(tokt, tailp):
    mesh = plsc.VectorSubcoreMesh(
        core_axis_name="c", subcore_axis_name="s",
        num_cores=NC, num_subcores=NS)
    return pl.kernel(
        _tr_body,
        out_type=jax.ShapeDtypeStruct((500000, PADE), jnp.float32),
        mesh=mesh,
        scratch_types=[
            pltpu.VMEM((EMBED, TB), jnp.float32),
            pltpu.VMEM((NRB, PADE), jnp.float32),
        ],
        compiler_params=pltpu.CompilerParams(
            use_tc_tiling_on_sc=True, needs_layout_passes=False),
    )(tokt, tailp)


def _sc_body(seq1d, seg1d, tok, comb, out,
             idx_v, seg_v, cidx_v, idxh_v, rows_v, crows_v, rows_o,
             sem_t, sem_c):
    c = lax.axis_index("c")
    s = lax.axis_index("s")
    wid = s * NC + c
    base = wid * NPW

    pltpu.sync_copy(seq1d.at[pl.ds(base, NPW)], idx_v)
    pltpu.sync_copy(seg1d.at[pl.ds(base, NPW)], seg_v)

    lane = lax.iota(jnp.int32, 16)

    @pl.loop(0, NPW // 16)
    def _cidx(g):
        off = g * 16
        pos = lax.rem(off + lane, SEQ)
        seg16 = seg_v[pl.ds(off, 16)]
        cidx_v[pl.ds(off, 16)] = seg16 * SEQ + pos
        idxh_v[pl.ds(off, 16)] = lax.shift_right_logical(
            idx_v[pl.ds(off, 16)], 1)

    @pl.loop(0, NCHUNK)
    def _chunk(ci):
        cb = ci * CH
        descs = []
        for k in range(KSUB):
            descs.append(pltpu.async_copy(
                tok.at[idxh_v.at[pl.ds(cb + k * SUB, SUB)]],
                rows_v.at[pl.ds(k * SUB, SUB)], sem_t))
            descs.append(pltpu.async_copy(
                comb.at[cidx_v.at[pl.ds(cb + k * SUB, SUB)]],
                crows_v.at[pl.ds(k * SUB, SUB)], sem_c))
        for d in descs:
            d.wait()

        @pl.loop(0, CH // 2)
        def _add(rp):
            for half in range(2):
                r = rp * 2 + half
                pidx = plsc.load_gather(idx_v, [lane * 0 + (cb + r)])
                oddf = lax.convert_element_type(pidx & 1, jnp.float32)
                for u in range(EMBED // 16):
                    sl = pl.ds(u * 16, 16)
                    lo = rows_v[r, sl]
                    hi = rows_v[r, pl.ds(EMBED + u * 16, 16)]
                    trow = lo + oddf * (hi - lo)
                    rows_o[rp, pl.ds(half * EMBED + u * 16, 16)] = (
                        trow + crows_v[r, sl])

        orow = pl.multiple_of((base + ci * CH) // 2, CH // 2)
        pltpu.sync_copy(rows_o, out.at[pl.ds(orow, CH // 2)])


@functools.partial(jax.jit, static_argnames=("interpret",))
def _sc_call(seq1d, seg1d, tok, comb, interpret=False):
    mesh = plsc.VectorSubcoreMesh(
        core_axis_name="c", subcore_axis_name="s",
        num_cores=NC, num_subcores=NS)
    return pl.kernel(
        _sc_body,
        out_type=jax.ShapeDtypeStruct((N // 2, PADE), jnp.float32),
        mesh=mesh,
        scratch_types=[
            pltpu.VMEM((NPW,), jnp.int32),
            pltpu.VMEM((NPW,), jnp.int32),
            pltpu.VMEM((NPW,), jnp.int32),
            pltpu.VMEM((NPW,), jnp.int32),
            pltpu.VMEM((CH, PADE), jnp.float32),
            pltpu.VMEM((CH, PADE), jnp.float32),
            pltpu.VMEM((CH // 2, PADE), jnp.float32),
            pltpu.SemaphoreType.DMA,
            pltpu.SemaphoreType.DMA,
        ],
        compiler_params=pltpu.CompilerParams(
            use_tc_tiling_on_sc=True, needs_layout_passes=False),
        interpret=interpret,
    )(seq1d, seg1d, tok, comb)


def kernel(sequence, segment_label, token_table, segment_table):
    b, s = sequence.shape
    seq1d = sequence.reshape(N).astype(jnp.int32)
    seg1d = segment_label.reshape(N).astype(jnp.int32)
    pe = jnp.asarray(_PE)
    comb = (segment_table[:, None, :] + pe[None, :, :]).reshape(3 * SEQ, EMBED)
    tokt = token_table.T
    tailp = jnp.pad(tokt[:, TAIL_T0:], ((0, 0), (0, PADE - TAIL_N)))
    tokp = _tr_call(tokt, tailp)
    combp = jnp.pad(comb, ((0, 0), (0, PADE - EMBED)))
    out = _sc_call(seq1d, seg1d, tokp, combp)
    return out.reshape(b, s, EMBED)
